# baseline (speedup 1.0000x reference)
import functools

import jax
import jax.numpy as jnp
from jax import lax
from jax.experimental import pallas as pl
from jax.experimental.pallas import tpu as pltpu
from jax.experimental.pallas import tpu_sc as plsc

NUM_CLASSES = 5
BACKGROUND = 4
GAMMA = 2.0
ALPHA_BG = 0.25
CLASS_W = 2.0
PT_CONF_W = 1.0
PT_COORD_W = 5.0
PAD_VALUE = -10000.0

B, Q, G, P = 8, 512, 128, 64
N_PAIR = B * G
N_ROW = B * Q


def _sc_assemble(srcm, mgtm, conf_t, coord_t, gtpt_t, flags_t, cls_t):
    info = plsc.get_sparse_core_info()
    nc, ns = info.num_cores, info.num_subcores
    nw = nc * ns
    pts = P // (nw // B)
    mesh = plsc.VectorSubcoreMesh(core_axis_name="c", subcore_axis_name="s")

    @functools.partial(
        pl.kernel,
        mesh=mesh,
        compiler_params=pltpu.CompilerParams(needs_layout_passes=False,
                                             skip_device_barrier=True),
        out_type=[
            jax.ShapeDtypeStruct((P, N_PAIR), jnp.float32),
            jax.ShapeDtypeStruct((NUM_CLASSES, N_PAIR), jnp.float32),
            jax.ShapeDtypeStruct((nw * 16,), jnp.float32),
            jax.ShapeDtypeStruct((nw * 16,), jnp.float32),
        ],
        scratch_types=[
            pltpu.VMEM((B, G), jnp.int32),
            pltpu.VMEM((B, G), jnp.int32),
            pltpu.VMEM((pts, Q), jnp.float32),
            pltpu.VMEM((pts, G), jnp.int32),
            pltpu.VMEM((pts, 2, Q), jnp.float32),
            pltpu.VMEM((pts, 2, G), jnp.float32),
            pltpu.VMEM((NUM_CLASSES, Q), jnp.float32),
            pltpu.VMEM((pts, G), jnp.float32),
            pltpu.VMEM((NUM_CLASSES, G), jnp.float32),
            pltpu.VMEM((16,), jnp.float32),
            pltpu.VMEM((16,), jnp.float32),
            pltpu.SemaphoreType.DMA,
            pltpu.SemaphoreType.DMA,
            pltpu.SemaphoreType.DMA,
        ],
    )
    def sc_kernel(src_hbm, mgt_hbm, conf_hbm, coord_hbm, gtpt_hbm, flags_hbm,
                  cls_hbm, q_out, mcls_out, l1_out, cnt_out,
                  src_all, mgt_all, conf_sl, flag_sl, coord_sl, gtpt_sl,
                  cls_sl, q_sl, mcls_sl, l1_v, cnt_v, sem, sem2, osem):
        wid = lax.axis_index("s") * nc + lax.axis_index("c")
        b = wid // (nw // B)
        p0 = (wid % (nw // B)) * pts

        pend = [
            pltpu.async_copy(src_hbm, src_all, sem),
            pltpu.async_copy(mgt_hbm, mgt_all, sem),
            pltpu.async_copy(conf_hbm.at[b, pl.ds(p0, pts)], conf_sl, sem),
            pltpu.async_copy(
                flags_hbm.at[pl.ds(p0, pts), pl.ds(b * G, G)], flag_sl, sem),
        ]
        pend2 = [
            pltpu.async_copy(coord_hbm.at[b, pl.ds(p0, pts)], coord_sl,
                             sem2),
            pltpu.async_copy(
                gtpt_hbm.at[pl.ds(p0, pts), :, pl.ds(b * G, G)], gtpt_sl,
                sem2),
        ]

        @pl.when(wid < B)
        def _():
            pltpu.async_copy(cls_hbm.at[:, wid % B], cls_sl, sem).wait()
        for cp in pend:
            cp.wait()

        opend = []
        for k in range(G // 16):
            sl = pl.ds(k * 16, 16)
            idx = src_all[b, sl]
            gti = mgt_all[b, sl]
            for i in range(pts):
                row = jnp.full((16,), i, jnp.int32)
                pv = plsc.load_gather(conf_sl, [row, idx])
                fv = plsc.load_gather(flag_sl, [row, gti])
                q_sl[i, sl] = jnp.where(fv != 0, pv, 1.0 - pv)
        opend.append(pltpu.async_copy(
            q_sl, q_out.at[pl.ds(p0, pts), pl.ds(b * G, G)], osem))

        for cp in pend2:
            cp.wait()
        l1 = jnp.zeros((16,), jnp.float32)
        cnt = jnp.zeros((16,), jnp.float32)
        for k in range(G // 16):
            sl = pl.ds(k * 16, 16)
            idx = src_all[b, sl]
            gti = mgt_all[b, sl]
            for i in range(pts):
                row = jnp.full((16,), i, jnp.int32)
                for c in range(2):
                    cv = jnp.full((16,), c, jnp.int32)
                    sp = plsc.load_gather(coord_sl, [row, cv, idx])
                    tp = plsc.load_gather(gtpt_sl, [row, cv, gti])
                    m = jnp.where(tp != PAD_VALUE, 1.0, 0.0)
                    l1 = l1 + jnp.abs(sp - tp) * m
                    cnt = cnt + m

        @pl.when(wid < B)
        def _():
            cb = wid % B
            for k in range(G // 16):
                sl = pl.ds(k * 16, 16)
                idx = src_all[cb, sl]
                for cc in range(NUM_CLASSES):
                    mcls_sl[cc, sl] = plsc.load_gather(
                        cls_sl, [jnp.full((16,), cc, jnp.int32), idx])
            pltpu.async_copy(
                mcls_sl, mcls_out.at[:, pl.ds(cb * G, G)], osem).wait()

        l1_v[...] = l1
        cnt_v[...] = cnt
        opend.append(pltpu.async_copy(l1_v, l1_out.at[pl.ds(wid * 16, 16)],
                                      osem))
        opend.append(pltpu.async_copy(cnt_v, cnt_out.at[pl.ds(wid * 16, 16)],
                                      osem))
        for cp in opend:
            cp.wait()

    return sc_kernel(srcm, mgtm, conf_t, coord_t, gtpt_t, flags_t, cls_t)


def _tc_body(cls_ref, src_r_ref, mgt_r_ref, gcls_r_ref, mcls_ref,
             q_ref, l1_ref, cnt_ref, out_ref):
    f32 = jnp.float32

    def lse0(x):
        m = jnp.max(x, axis=0, keepdims=True)
        return jnp.log(jnp.sum(jnp.exp(x - m), axis=0, keepdims=True)) + m

    eye = (lax.broadcasted_iota(jnp.int32, (G, G), 0)
           == lax.broadcasted_iota(jnp.int32, (G, G), 1))

    def to_col(vrow):
        return jnp.sum(jnp.where(eye, vrow, 0), axis=1, keepdims=True)

    x = cls_ref[...]
    ls4 = x[BACKGROUND:BACKGROUND + 1] - lse0(x)
    p4 = jnp.exp(ls4)
    base_sum = jnp.sum(-(1.0 - ALPHA_BG) * (1.0 - p4) * (1.0 - p4) * ls4)

    corr = f32(0.0)
    for b in range(B):
        cs = pl.ds(b * G, G)
        sr = src_r_ref[:, cs]
        sc = to_col(sr)
        gi = lax.broadcasted_iota(jnp.int32, (G, G), 0)
        gj = lax.broadcasted_iota(jnp.int32, (G, G), 1)
        later = jnp.where((sc == sr) & (gi > gj), 1, 0)
        conflict = jnp.max(later, axis=0, keepdims=True)
        winner = (conflict == 0).astype(f32)

        mr = mgt_r_ref[:, cs]
        gmat = mr == gi
        gcls = to_col(gcls_r_ref[:, cs]).astype(f32)
        tcls = jnp.sum(jnp.where(gmat, gcls, 0.0), axis=0, keepdims=True)

        xm = mcls_ref[:, cs]
        lsoft = xm - lse0(xm)
        lane0 = lax.broadcasted_iota(jnp.int32, (NUM_CLASSES, G), 0)
        onehot = (lane0.astype(f32) == tcls).astype(f32)
        logp_t = jnp.sum(lsoft * onehot, axis=0, keepdims=True)
        p_t = jnp.exp(logp_t)
        alpha = jnp.where(tcls == 0.0, ALPHA_BG, 1.0 - ALPHA_BG)
        loss_new = -alpha * (1.0 - p_t) * (1.0 - p_t) * logp_t
        ls4m = lsoft[BACKGROUND:BACKGROUND + 1, :]
        p4m = jnp.exp(ls4m)
        loss_old = -(1.0 - ALPHA_BG) * (1.0 - p4m) * (1.0 - p4m) * ls4m
        corr = corr + jnp.sum(winner * (loss_new - loss_old))

    class_loss = CLASS_W * (base_sum + corr) / f32(N_ROW)

    conf_loss = PT_CONF_W * (-jnp.sum(jnp.log(q_ref[...])) / f32(N_PAIR * P))

    l1s = jnp.sum(l1_ref[...])
    cnts = jnp.sum(cnt_ref[...])
    coord_loss = PT_COORD_W * l1s / jnp.maximum(cnts, 1.0)

    lane = lax.broadcasted_iota(jnp.int32, (1, 128), 1)
    out = (jnp.where(lane == 0, class_loss, 0.0)
           + jnp.where(lane == 1, conf_loss, 0.0)
           + jnp.where(lane == 2, coord_loss, 0.0))
    out_ref[...] = out.astype(f32)


def _tc_losses(cls_t, src_row, mgt_row, gcls_row, mcls, q, l1_part, cnt_part):
    return pl.pallas_call(
        _tc_body,
        out_shape=jax.ShapeDtypeStruct((1, 128), jnp.float32),
    )(cls_t, src_row, mgt_row, gcls_row, mcls, q, l1_part, cnt_part)


def kernel(cls_pred, point_coord_pred, point_confidence_pred,
           matched_src_idx, matched_gt_idx, gt_class, gt_points,
           gt_pt_padding_flags, gt_num):
    i32 = jnp.int32
    srcm = matched_src_idx.astype(i32)
    mgtm = matched_gt_idx.astype(i32)

    conf_t = jnp.transpose(point_confidence_pred, (0, 2, 1))
    coord_t = jnp.transpose(point_coord_pred, (0, 2, 3, 1))
    gtpt_t = jnp.transpose(gt_points, (1, 2, 0))
    flags_t = jnp.transpose(gt_pt_padding_flags.astype(i32), (1, 0))
    cls_t = jnp.transpose(cls_pred, (2, 0, 1))

    q, mcls, l1_part, cnt_part = _sc_assemble(
        srcm, mgtm, conf_t, coord_t, gtpt_t, flags_t, cls_t)

    out = _tc_losses(cls_t, srcm.reshape(1, N_PAIR), mgtm.reshape(1, N_PAIR),
                     gt_class.astype(i32).reshape(1, N_PAIR), mcls, q,
                     l1_part.reshape(4, 128), cnt_part.reshape(4, 128))
    return (out[0, 0], out[0, 1], out[0, 2])


# speedup vs baseline: 1.0965x; 1.0965x over previous
import functools

import jax
import jax.numpy as jnp
from jax import lax
from jax.experimental import pallas as pl
from jax.experimental.pallas import tpu as pltpu
from jax.experimental.pallas import tpu_sc as plsc

NUM_CLASSES = 5
BACKGROUND = 4
GAMMA = 2.0
ALPHA_BG = 0.25
CLASS_W = 2.0
PT_CONF_W = 1.0
PT_COORD_W = 5.0
PAD_VALUE = -10000.0

B, Q, G, P = 8, 512, 128, 64
N_PAIR = B * G
N_ROW = B * Q


def _sc_assemble(srcm, mgtm, conf_t, coord_t, gtpt_t, flags_t, cls_t):
    info = plsc.get_sparse_core_info()
    nc, ns = info.num_cores, info.num_subcores
    nw = nc * ns
    pts = P // (nw // B)
    mesh = plsc.VectorSubcoreMesh(core_axis_name="c", subcore_axis_name="s")

    @functools.partial(
        pl.kernel,
        mesh=mesh,
        compiler_params=pltpu.CompilerParams(needs_layout_passes=False,
                                             skip_device_barrier=True),
        out_type=[
            jax.ShapeDtypeStruct((P, N_PAIR), jnp.float32),
            jax.ShapeDtypeStruct((NUM_CLASSES, N_PAIR), jnp.float32),
            jax.ShapeDtypeStruct((nw * 16,), jnp.float32),
            jax.ShapeDtypeStruct((nw * 16,), jnp.float32),
        ],
        scratch_types=[
            pltpu.VMEM((B, G), jnp.int32),
            pltpu.VMEM((B, G), jnp.int32),
            pltpu.VMEM((pts, Q), jnp.float32),
            pltpu.VMEM((pts, G), jnp.int32),
            pltpu.VMEM((pts, 2, Q), jnp.float32),
            pltpu.VMEM((pts, 2, G), jnp.float32),
            pltpu.VMEM((NUM_CLASSES, Q), jnp.float32),
            pltpu.VMEM((pts, G), jnp.float32),
            pltpu.VMEM((NUM_CLASSES, G), jnp.float32),
            pltpu.VMEM((16,), jnp.float32),
            pltpu.VMEM((16,), jnp.float32),
            pltpu.SemaphoreType.DMA,
            pltpu.SemaphoreType.DMA,
            pltpu.SemaphoreType.DMA,
        ],
    )
    def sc_kernel(src_hbm, mgt_hbm, conf_hbm, coord_hbm, gtpt_hbm, flags_hbm,
                  cls_hbm, q_out, mcls_out, l1_out, cnt_out,
                  src_all, mgt_all, conf_sl, flag_sl, coord_sl, gtpt_sl,
                  cls_sl, q_sl, mcls_sl, l1_v, cnt_v, sem, sem2, osem):
        wid = lax.axis_index("s") * nc + lax.axis_index("c")
        b = wid // (nw // B)
        p0 = (wid % (nw // B)) * pts

        pend = [
            pltpu.async_copy(src_hbm, src_all, sem),
            pltpu.async_copy(mgt_hbm, mgt_all, sem),
            pltpu.async_copy(conf_hbm.at[b, pl.ds(p0, pts)], conf_sl, sem),
            pltpu.async_copy(
                flags_hbm.at[pl.ds(p0, pts), pl.ds(b * G, G)], flag_sl, sem),
        ]
        pend2 = [
            pltpu.async_copy(coord_hbm.at[b, pl.ds(p0, pts)], coord_sl,
                             sem2),
            pltpu.async_copy(
                gtpt_hbm.at[pl.ds(p0, pts), :, pl.ds(b * G, G)], gtpt_sl,
                sem2),
        ]

        @pl.when(wid < B)
        def _():
            pltpu.async_copy(cls_hbm.at[:, wid % B], cls_sl, sem).wait()
        for cp in pend:
            cp.wait()

        lane16 = lax.broadcasted_iota(jnp.int32, (16,), 0)

        opend = []
        def qchunk(k, _):
            col = lane16 + k * 16
            idx = plsc.load_gather(src_all, [jnp.full((16,), b, jnp.int32),
                                             col])
            gti = plsc.load_gather(mgt_all, [jnp.full((16,), b, jnp.int32),
                                             col])
            for i in range(pts):
                row = jnp.full((16,), i, jnp.int32)
                pv = plsc.load_gather(conf_sl, [row, idx])
                fv = plsc.load_gather(flag_sl, [row, gti])
                plsc.store_scatter(q_sl, [row, col],
                                   jnp.where(fv != 0, pv, 1.0 - pv))
            return 0

        lax.fori_loop(0, G // 16, qchunk, 0)
        opend.append(pltpu.async_copy(
            q_sl, q_out.at[pl.ds(p0, pts), pl.ds(b * G, G)], osem))

        for cp in pend2:
            cp.wait()

        def cchunk(k, carry):
            l1, cnt = carry
            col = lane16 + k * 16
            idx = plsc.load_gather(src_all, [jnp.full((16,), b, jnp.int32),
                                             col])
            gti = plsc.load_gather(mgt_all, [jnp.full((16,), b, jnp.int32),
                                             col])
            for i in range(pts):
                row = jnp.full((16,), i, jnp.int32)
                for c in range(2):
                    cv = jnp.full((16,), c, jnp.int32)
                    sp = plsc.load_gather(coord_sl, [row, cv, idx])
                    tp = plsc.load_gather(gtpt_sl, [row, cv, gti])
                    m = jnp.where(tp != PAD_VALUE, 1.0, 0.0)
                    l1 = l1 + jnp.abs(sp - tp) * m
                    cnt = cnt + m
            return (l1, cnt)

        l1, cnt = lax.fori_loop(0, G // 16, cchunk,
                                (jnp.zeros((16,), jnp.float32),
                                 jnp.zeros((16,), jnp.float32)))

        @pl.when(wid < B)
        def _():
            cb = wid % B

            def mchunk(k, _):
                col = lane16 + k * 16
                idx = plsc.load_gather(
                    src_all, [jnp.full((16,), cb, jnp.int32), col])
                for cc in range(NUM_CLASSES):
                    ccv = jnp.full((16,), cc, jnp.int32)
                    plsc.store_scatter(
                        mcls_sl, [ccv, col],
                        plsc.load_gather(cls_sl, [ccv, idx]))
                return 0

            lax.fori_loop(0, G // 16, mchunk, 0)
            pltpu.async_copy(
                mcls_sl, mcls_out.at[:, pl.ds(cb * G, G)], osem).wait()

        l1_v[...] = l1
        cnt_v[...] = cnt
        opend.append(pltpu.async_copy(l1_v, l1_out.at[pl.ds(wid * 16, 16)],
                                      osem))
        opend.append(pltpu.async_copy(cnt_v, cnt_out.at[pl.ds(wid * 16, 16)],
                                      osem))
        for cp in opend:
            cp.wait()

    return sc_kernel(srcm, mgtm, conf_t, coord_t, gtpt_t, flags_t, cls_t)


def _tc_body(cls_ref, src_r_ref, mgt_r_ref, gcls_r_ref, mcls_ref,
             q_ref, l1_ref, cnt_ref, cls_out, conf_out, coord_out):
    f32 = jnp.float32

    def lse0(x):
        m = jnp.max(x, axis=0, keepdims=True)
        return jnp.log(jnp.sum(jnp.exp(x - m), axis=0, keepdims=True)) + m

    eye = (lax.broadcasted_iota(jnp.int32, (G, G), 0)
           == lax.broadcasted_iota(jnp.int32, (G, G), 1))

    def to_col(vrow):
        return jnp.sum(jnp.where(eye, vrow, 0), axis=1, keepdims=True)

    x = cls_ref[...]
    ls4 = x[BACKGROUND:BACKGROUND + 1] - lse0(x)
    p4 = jnp.exp(ls4)
    base_sum = jnp.sum(-(1.0 - ALPHA_BG) * (1.0 - p4) * (1.0 - p4) * ls4)

    corr = f32(0.0)
    for b in range(B):
        cs = pl.ds(b * G, G)
        sr = src_r_ref[:, cs]
        sc = to_col(sr)
        gi = lax.broadcasted_iota(jnp.int32, (G, G), 0)
        gj = lax.broadcasted_iota(jnp.int32, (G, G), 1)
        later = jnp.where((sc == sr) & (gi > gj), 1, 0)
        conflict = jnp.max(later, axis=0, keepdims=True)
        winner = (conflict == 0).astype(f32)

        mr = mgt_r_ref[:, cs]
        gmat = mr == gi
        gcls = to_col(gcls_r_ref[:, cs]).astype(f32)
        tcls = jnp.sum(jnp.where(gmat, gcls, 0.0), axis=0, keepdims=True)

        xm = mcls_ref[:, cs]
        lsoft = xm - lse0(xm)
        lane0 = lax.broadcasted_iota(jnp.int32, (NUM_CLASSES, G), 0)
        onehot = (lane0.astype(f32) == tcls).astype(f32)
        logp_t = jnp.sum(lsoft * onehot, axis=0, keepdims=True)
        p_t = jnp.exp(logp_t)
        alpha = jnp.where(tcls == 0.0, ALPHA_BG, 1.0 - ALPHA_BG)
        loss_new = -alpha * (1.0 - p_t) * (1.0 - p_t) * logp_t
        ls4m = lsoft[BACKGROUND:BACKGROUND + 1, :]
        p4m = jnp.exp(ls4m)
        loss_old = -(1.0 - ALPHA_BG) * (1.0 - p4m) * (1.0 - p4m) * ls4m
        corr = corr + jnp.sum(winner * (loss_new - loss_old))

    class_loss = CLASS_W * (base_sum + corr) / f32(N_ROW)

    conf_loss = PT_CONF_W * (-jnp.sum(jnp.log(q_ref[...])) / f32(N_PAIR * P))

    l1s = jnp.sum(l1_ref[...])
    cnts = jnp.sum(cnt_ref[...])
    coord_loss = PT_COORD_W * l1s / jnp.maximum(cnts, 1.0)

    cls_out[...] = jnp.reshape(class_loss, (1, 1))
    conf_out[...] = jnp.reshape(conf_loss, (1, 1))
    coord_out[...] = jnp.reshape(coord_loss, (1, 1))


def _tc_losses(cls_t, src_row, mgt_row, gcls_row, mcls, q, l1_part, cnt_part):
    s = jax.ShapeDtypeStruct((1, 1), jnp.float32)
    return pl.pallas_call(
        _tc_body,
        out_shape=(s, s, s),
    )(cls_t, src_row, mgt_row, gcls_row, mcls, q, l1_part, cnt_part)


def kernel(cls_pred, point_coord_pred, point_confidence_pred,
           matched_src_idx, matched_gt_idx, gt_class, gt_points,
           gt_pt_padding_flags, gt_num):
    i32 = jnp.int32
    srcm = matched_src_idx.astype(i32)
    mgtm = matched_gt_idx.astype(i32)

    conf_t = jnp.transpose(point_confidence_pred, (0, 2, 1))
    coord_t = jnp.transpose(point_coord_pred, (0, 2, 3, 1))
    gtpt_t = jnp.transpose(gt_points, (1, 2, 0))
    flags_t = jnp.transpose(gt_pt_padding_flags.astype(i32), (1, 0))
    cls_t = jnp.transpose(cls_pred, (2, 0, 1))

    q, mcls, l1_part, cnt_part = _sc_assemble(
        srcm, mgtm, conf_t, coord_t, gtpt_t, flags_t, cls_t)

    cl, co, cd = _tc_losses(cls_t, srcm.reshape(1, N_PAIR),
                            mgtm.reshape(1, N_PAIR),
                            gt_class.astype(i32).reshape(1, N_PAIR), mcls, q,
                            l1_part.reshape(4, 128), cnt_part.reshape(4, 128))
    return (cl.reshape(()), co.reshape(()), cd.reshape(()))


# speedup vs baseline: 1.3345x; 1.2170x over previous
import functools

import jax
import jax.numpy as jnp
from jax import lax
from jax.experimental import pallas as pl
from jax.experimental.pallas import tpu as pltpu
from jax.experimental.pallas import tpu_sc as plsc

NUM_CLASSES = 5
BACKGROUND = 4
GAMMA = 2.0
ALPHA_BG = 0.25
CLASS_W = 2.0
PT_CONF_W = 1.0
PT_COORD_W = 5.0
PAD_VALUE = -10000.0

B, Q, G, P = 8, 512, 128, 64
N_PAIR = B * G
N_ROW = B * Q


def _sc_assemble(srcm, mgtm, conf_t, coord_t, gtpt_t, flags_t, cls_t):
    info = plsc.get_sparse_core_info()
    nc, ns = info.num_cores, info.num_subcores
    nw = nc * ns
    pts = P // (nw // B)
    mesh = plsc.VectorSubcoreMesh(core_axis_name="c", subcore_axis_name="s")

    @functools.partial(
        pl.kernel,
        mesh=mesh,
        compiler_params=pltpu.CompilerParams(needs_layout_passes=False,
                                             skip_device_barrier=True),
        out_type=[
            jax.ShapeDtypeStruct((P, N_PAIR), jnp.float32),
            jax.ShapeDtypeStruct((NUM_CLASSES, N_PAIR), jnp.float32),
            jax.ShapeDtypeStruct((nw * 16,), jnp.float32),
            jax.ShapeDtypeStruct((nw * 16,), jnp.float32),
        ],
        scratch_types=[
            pltpu.VMEM((B, G), jnp.int32),
            pltpu.VMEM((B, G), jnp.int32),
            pltpu.VMEM((pts, Q), jnp.float32),
            pltpu.VMEM((pts, G), jnp.int32),
            pltpu.VMEM((pts, 2, Q), jnp.float32),
            pltpu.VMEM((pts, 2, G), jnp.float32),
            pltpu.VMEM((NUM_CLASSES, Q), jnp.float32),
            pltpu.VMEM((pts, G), jnp.float32),
            pltpu.VMEM((NUM_CLASSES, G), jnp.float32),
            pltpu.VMEM((16,), jnp.float32),
            pltpu.VMEM((16,), jnp.float32),
            pltpu.SemaphoreType.DMA,
            pltpu.SemaphoreType.DMA,
            pltpu.SemaphoreType.DMA,
        ],
    )
    def sc_kernel(src_hbm, mgt_hbm, conf_hbm, coord_hbm, gtpt_hbm, flags_hbm,
                  cls_hbm, q_out, mcls_out, l1_out, cnt_out,
                  src_all, mgt_all, conf_sl, flag_sl, coord_sl, gtpt_sl,
                  cls_sl, q_sl, mcls_sl, l1_v, cnt_v, sem, sem2, osem):
        wid = lax.axis_index("s") * nc + lax.axis_index("c")
        b = wid // (nw // B)
        p0 = (wid % (nw // B)) * pts

        pend = [
            pltpu.async_copy(src_hbm, src_all, sem),
            pltpu.async_copy(mgt_hbm, mgt_all, sem),
            pltpu.async_copy(conf_hbm.at[b, pl.ds(p0, pts)], conf_sl, sem),
            pltpu.async_copy(
                flags_hbm.at[pl.ds(p0, pts), pl.ds(b * G, G)], flag_sl, sem),
        ]
        pend2 = [
            pltpu.async_copy(coord_hbm.at[b, pl.ds(p0, pts)], coord_sl,
                             sem2),
            pltpu.async_copy(
                gtpt_hbm.at[pl.ds(p0, pts), :, pl.ds(b * G, G)], gtpt_sl,
                sem2),
        ]

        @pl.when(wid < B)
        def _():
            pltpu.async_copy(cls_hbm.at[:, wid % B], cls_sl, sem).wait()
        for cp in pend:
            cp.wait()

        lane16 = lax.broadcasted_iota(jnp.int32, (16,), 0)

        opend = []
        def qchunk(k, _):
            col = lane16 + k * 16
            idx = plsc.load_gather(src_all, [jnp.full((16,), b, jnp.int32),
                                             col])
            gti = plsc.load_gather(mgt_all, [jnp.full((16,), b, jnp.int32),
                                             col])

            def qrow(i, _):
                row = jnp.full((16,), 0, jnp.int32) + i
                pv = plsc.load_gather(conf_sl, [row, idx])
                fv = plsc.load_gather(flag_sl, [row, gti])
                plsc.store_scatter(q_sl, [row, col],
                                   jnp.where(fv != 0, pv, 1.0 - pv))
                return 0

            return lax.fori_loop(0, pts, qrow, 0)

        lax.fori_loop(0, G // 16, qchunk, 0)
        opend.append(pltpu.async_copy(
            q_sl, q_out.at[pl.ds(p0, pts), pl.ds(b * G, G)], osem))

        for cp in pend2:
            cp.wait()

        def cchunk(k, carry):
            col = lane16 + k * 16
            idx = plsc.load_gather(src_all, [jnp.full((16,), b, jnp.int32),
                                             col])
            gti = plsc.load_gather(mgt_all, [jnp.full((16,), b, jnp.int32),
                                             col])

            def crow(i, carry2):
                l1, cnt = carry2
                row = jnp.full((16,), 0, jnp.int32) + i
                for c in range(2):
                    cv = jnp.full((16,), c, jnp.int32)
                    sp = plsc.load_gather(coord_sl, [row, cv, idx])
                    tp = plsc.load_gather(gtpt_sl, [row, cv, gti])
                    m = jnp.where(tp != PAD_VALUE, 1.0, 0.0)
                    l1 = l1 + jnp.abs(sp - tp) * m
                    cnt = cnt + m
                return (l1, cnt)

            return lax.fori_loop(0, pts, crow, carry)

        l1, cnt = lax.fori_loop(0, G // 16, cchunk,
                                (jnp.zeros((16,), jnp.float32),
                                 jnp.zeros((16,), jnp.float32)))

        @pl.when(wid < B)
        def _():
            cb = wid % B

            def mchunk(k, _):
                col = lane16 + k * 16
                idx = plsc.load_gather(
                    src_all, [jnp.full((16,), cb, jnp.int32), col])
                for cc in range(NUM_CLASSES):
                    ccv = jnp.full((16,), cc, jnp.int32)
                    plsc.store_scatter(
                        mcls_sl, [ccv, col],
                        plsc.load_gather(cls_sl, [ccv, idx]))
                return 0

            lax.fori_loop(0, G // 16, mchunk, 0)
            pltpu.async_copy(
                mcls_sl, mcls_out.at[:, pl.ds(cb * G, G)], osem).wait()

        l1_v[...] = l1
        cnt_v[...] = cnt
        opend.append(pltpu.async_copy(l1_v, l1_out.at[pl.ds(wid * 16, 16)],
                                      osem))
        opend.append(pltpu.async_copy(cnt_v, cnt_out.at[pl.ds(wid * 16, 16)],
                                      osem))
        for cp in opend:
            cp.wait()

    return sc_kernel(srcm, mgtm, conf_t, coord_t, gtpt_t, flags_t, cls_t)


def _tc_body(cls_ref, src_r_ref, mgt_r_ref, gcls_r_ref, mcls_ref,
             q_ref, l1_ref, cnt_ref, cls_out, conf_out, coord_out):
    f32 = jnp.float32

    def lse0(x):
        m = jnp.max(x, axis=0, keepdims=True)
        return jnp.log(jnp.sum(jnp.exp(x - m), axis=0, keepdims=True)) + m

    eye = (lax.broadcasted_iota(jnp.int32, (G, G), 0)
           == lax.broadcasted_iota(jnp.int32, (G, G), 1))

    def to_col(vrow):
        return jnp.sum(jnp.where(eye, vrow, 0), axis=1, keepdims=True)

    x = cls_ref[...]
    ls4 = x[BACKGROUND:BACKGROUND + 1] - lse0(x)
    p4 = jnp.exp(ls4)
    base_sum = jnp.sum(-(1.0 - ALPHA_BG) * (1.0 - p4) * (1.0 - p4) * ls4)

    corr = f32(0.0)
    for b in range(B):
        cs = pl.ds(b * G, G)
        sr = src_r_ref[:, cs]
        sc = to_col(sr)
        gi = lax.broadcasted_iota(jnp.int32, (G, G), 0)
        gj = lax.broadcasted_iota(jnp.int32, (G, G), 1)
        later = jnp.where((sc == sr) & (gi > gj), 1, 0)
        conflict = jnp.max(later, axis=0, keepdims=True)
        winner = (conflict == 0).astype(f32)

        mr = mgt_r_ref[:, cs]
        gmat = mr == gi
        gcls = to_col(gcls_r_ref[:, cs]).astype(f32)
        tcls = jnp.sum(jnp.where(gmat, gcls, 0.0), axis=0, keepdims=True)

        xm = mcls_ref[:, cs]
        lsoft = xm - lse0(xm)
        lane0 = lax.broadcasted_iota(jnp.int32, (NUM_CLASSES, G), 0)
        onehot = (lane0.astype(f32) == tcls).astype(f32)
        logp_t = jnp.sum(lsoft * onehot, axis=0, keepdims=True)
        p_t = jnp.exp(logp_t)
        alpha = jnp.where(tcls == 0.0, ALPHA_BG, 1.0 - ALPHA_BG)
        loss_new = -alpha * (1.0 - p_t) * (1.0 - p_t) * logp_t
        ls4m = lsoft[BACKGROUND:BACKGROUND + 1, :]
        p4m = jnp.exp(ls4m)
        loss_old = -(1.0 - ALPHA_BG) * (1.0 - p4m) * (1.0 - p4m) * ls4m
        corr = corr + jnp.sum(winner * (loss_new - loss_old))

    class_loss = CLASS_W * (base_sum + corr) / f32(N_ROW)

    conf_loss = PT_CONF_W * (-jnp.sum(jnp.log(q_ref[...])) / f32(N_PAIR * P))

    l1s = jnp.sum(l1_ref[...])
    cnts = jnp.sum(cnt_ref[...])
    coord_loss = PT_COORD_W * l1s / jnp.maximum(cnts, 1.0)

    cls_out[...] = jnp.reshape(class_loss, (1, 1))
    conf_out[...] = jnp.reshape(conf_loss, (1, 1))
    coord_out[...] = jnp.reshape(coord_loss, (1, 1))


def _tc_losses(cls_t, src_row, mgt_row, gcls_row, mcls, q, l1_part, cnt_part):
    s = jax.ShapeDtypeStruct((1, 1), jnp.float32)
    return pl.pallas_call(
        _tc_body,
        out_shape=(s, s, s),
    )(cls_t, src_row, mgt_row, gcls_row, mcls, q, l1_part, cnt_part)


def kernel(cls_pred, point_coord_pred, point_confidence_pred,
           matched_src_idx, matched_gt_idx, gt_class, gt_points,
           gt_pt_padding_flags, gt_num):
    i32 = jnp.int32
    srcm = matched_src_idx.astype(i32)
    mgtm = matched_gt_idx.astype(i32)

    conf_t = jnp.transpose(point_confidence_pred, (0, 2, 1))
    coord_t = jnp.transpose(point_coord_pred, (0, 2, 3, 1))
    gtpt_t = jnp.transpose(gt_points, (1, 2, 0))
    flags_t = jnp.transpose(gt_pt_padding_flags.astype(i32), (1, 0))
    cls_t = jnp.transpose(cls_pred, (2, 0, 1))

    q, mcls, l1_part, cnt_part = _sc_assemble(
        srcm, mgtm, conf_t, coord_t, gtpt_t, flags_t, cls_t)

    cl, co, cd = _tc_losses(cls_t, srcm.reshape(1, N_PAIR),
                            mgtm.reshape(1, N_PAIR),
                            gt_class.astype(i32).reshape(1, N_PAIR), mcls, q,
                            l1_part.reshape(4, 128), cnt_part.reshape(4, 128))
    return (cl.reshape(()), co.reshape(()), cd.reshape(()))


# speedup vs baseline: 1.3598x; 1.0190x over previous
import functools

import jax
import jax.numpy as jnp
from jax import lax
from jax.experimental import pallas as pl
from jax.experimental.pallas import tpu as pltpu
from jax.experimental.pallas import tpu_sc as plsc

NUM_CLASSES = 5
BACKGROUND = 4
GAMMA = 2.0
ALPHA_BG = 0.25
CLASS_W = 2.0
PT_CONF_W = 1.0
PT_COORD_W = 5.0
PAD_VALUE = -10000.0

B, Q, G, P = 8, 512, 128, 64
N_PAIR = B * G
N_ROW = B * Q


def _sc_assemble(srcm, mgtm, conf_t, coord_t, gtpt_t, flags_t, cls_t):
    info = plsc.get_sparse_core_info()
    nc, ns = info.num_cores, info.num_subcores
    nw = nc * ns
    pts = P // (nw // B)
    mesh = plsc.VectorSubcoreMesh(core_axis_name="c", subcore_axis_name="s")

    @functools.partial(
        pl.kernel,
        mesh=mesh,
        compiler_params=pltpu.CompilerParams(needs_layout_passes=False,
                                             skip_device_barrier=True),
        out_type=[
            jax.ShapeDtypeStruct((P, N_PAIR), jnp.float32),
            jax.ShapeDtypeStruct((NUM_CLASSES, N_PAIR), jnp.float32),
            jax.ShapeDtypeStruct((nw * 16,), jnp.float32),
            jax.ShapeDtypeStruct((nw * 16,), jnp.float32),
        ],
        scratch_types=[
            pltpu.VMEM((B, G), jnp.int32),
            pltpu.VMEM((B, G), jnp.int32),
            pltpu.VMEM((pts, Q), jnp.float32),
            pltpu.VMEM((pts, G), jnp.int32),
            pltpu.VMEM((pts, 2, Q), jnp.float32),
            pltpu.VMEM((pts, 2, G), jnp.float32),
            pltpu.VMEM((NUM_CLASSES, Q), jnp.float32),
            pltpu.VMEM((pts, G), jnp.float32),
            pltpu.VMEM((NUM_CLASSES, G), jnp.float32),
            pltpu.VMEM((16,), jnp.float32),
            pltpu.VMEM((16,), jnp.float32),
            pltpu.SemaphoreType.DMA,
            pltpu.SemaphoreType.DMA,
            pltpu.SemaphoreType.DMA,
        ],
    )
    def sc_kernel(src_hbm, mgt_hbm, conf_hbm, coord_hbm, gtpt_hbm, flags_hbm,
                  cls_hbm, q_out, mcls_out, l1_out, cnt_out,
                  src_all, mgt_all, conf_sl, flag_sl, coord_sl, gtpt_sl,
                  cls_sl, q_sl, mcls_sl, l1_v, cnt_v, sem, sem2, osem):
        wid = lax.axis_index("s") * nc + lax.axis_index("c")
        b = wid // (nw // B)
        p0 = (wid % (nw // B)) * pts

        pend = [
            pltpu.async_copy(src_hbm, src_all, sem),
            pltpu.async_copy(mgt_hbm, mgt_all, sem),
            pltpu.async_copy(conf_hbm.at[b, pl.ds(p0, pts)], conf_sl, sem),
            pltpu.async_copy(
                flags_hbm.at[pl.ds(p0, pts), pl.ds(b * G, G)], flag_sl, sem),
        ]
        pend2 = [
            pltpu.async_copy(coord_hbm.at[b, pl.ds(p0, pts)], coord_sl,
                             sem2),
            pltpu.async_copy(
                gtpt_hbm.at[pl.ds(p0, pts), :, pl.ds(b * G, G)], gtpt_sl,
                sem2),
        ]

        @pl.when(wid < B)
        def _():
            pltpu.async_copy(cls_hbm.at[:, wid % B], cls_sl, sem).wait()
        for cp in pend:
            cp.wait()

        lane16 = lax.broadcasted_iota(jnp.int32, (16,), 0)

        opend = []
        def qchunk(k, _):
            col = lane16 + k * 16
            idx = plsc.load_gather(src_all, [jnp.full((16,), b, jnp.int32),
                                             col])
            gti = plsc.load_gather(mgt_all, [jnp.full((16,), b, jnp.int32),
                                             col])

            def qrow(i, _):
                row = jnp.full((16,), 0, jnp.int32) + i
                pv = plsc.load_gather(conf_sl, [row, idx])
                fv = plsc.load_gather(flag_sl, [row, gti])
                plsc.store_scatter(q_sl, [row, col],
                                   jnp.where(fv != 0, pv, 1.0 - pv))
                return 0

            return lax.fori_loop(0, pts, qrow, 0)

        lax.fori_loop(0, G // 16, qchunk, 0)
        opend.append(pltpu.async_copy(
            q_sl, q_out.at[pl.ds(p0, pts), pl.ds(b * G, G)], osem))

        for cp in pend2:
            cp.wait()

        def cchunk(k, carry):
            col = lane16 + k * 16
            idx = plsc.load_gather(src_all, [jnp.full((16,), b, jnp.int32),
                                             col])
            gti = plsc.load_gather(mgt_all, [jnp.full((16,), b, jnp.int32),
                                             col])

            def crow(i, carry2):
                l1, cnt = carry2
                row = jnp.full((16,), 0, jnp.int32) + i
                for c in range(2):
                    cv = jnp.full((16,), c, jnp.int32)
                    sp = plsc.load_gather(coord_sl, [row, cv, idx])
                    tp = plsc.load_gather(gtpt_sl, [row, cv, gti])
                    m = jnp.where(tp != PAD_VALUE, 1.0, 0.0)
                    l1 = l1 + jnp.abs(sp - tp) * m
                    cnt = cnt + m
                return (l1, cnt)

            return lax.fori_loop(0, pts, crow, carry)

        l1, cnt = lax.fori_loop(0, G // 16, cchunk,
                                (jnp.zeros((16,), jnp.float32),
                                 jnp.zeros((16,), jnp.float32)))

        @pl.when(wid < B)
        def _():
            cb = wid % B

            def mchunk(k, _):
                col = lane16 + k * 16
                idx = plsc.load_gather(
                    src_all, [jnp.full((16,), cb, jnp.int32), col])
                for cc in range(NUM_CLASSES):
                    ccv = jnp.full((16,), cc, jnp.int32)
                    plsc.store_scatter(
                        mcls_sl, [ccv, col],
                        plsc.load_gather(cls_sl, [ccv, idx]))
                return 0

            lax.fori_loop(0, G // 16, mchunk, 0)
            pltpu.async_copy(
                mcls_sl, mcls_out.at[:, pl.ds(cb * G, G)], osem).wait()

        l1_v[...] = l1
        cnt_v[...] = cnt
        opend.append(pltpu.async_copy(l1_v, l1_out.at[pl.ds(wid * 16, 16)],
                                      osem))
        opend.append(pltpu.async_copy(cnt_v, cnt_out.at[pl.ds(wid * 16, 16)],
                                      osem))
        for cp in opend:
            cp.wait()

    return sc_kernel(srcm, mgtm, conf_t, coord_t, gtpt_t, flags_t, cls_t)


def _lse0(x):
    m = jnp.max(x, axis=0, keepdims=True)
    return jnp.log(jnp.sum(jnp.exp(x - m), axis=0, keepdims=True)) + m


def _tc_a_body(cls_ref, src_r_ref, mgt_r_ref, gcls_r_ref,
               base_out, win_out, tcls_out):
    f32 = jnp.float32

    eye = (lax.broadcasted_iota(jnp.int32, (G, G), 0)
           == lax.broadcasted_iota(jnp.int32, (G, G), 1))

    def to_col(vrow):
        return jnp.sum(jnp.where(eye, vrow, 0), axis=1, keepdims=True)

    x = cls_ref[...]
    ls4 = x[BACKGROUND:BACKGROUND + 1] - _lse0(x)
    p4 = jnp.exp(ls4)
    base_sum = jnp.sum(-(1.0 - ALPHA_BG) * (1.0 - p4) * (1.0 - p4) * ls4)
    base_out[...] = jnp.reshape(base_sum, (1, 1))

    gi = lax.broadcasted_iota(jnp.int32, (G, G), 0)
    gj = lax.broadcasted_iota(jnp.int32, (G, G), 1)
    for b in range(B):
        cs = pl.ds(b * G, G)
        sr = src_r_ref[:, cs]
        sc = to_col(sr)
        later = jnp.where((sc == sr) & (gi > gj), 1, 0)
        conflict = jnp.max(later, axis=0, keepdims=True)
        win_out[b, :] = (conflict == 0).astype(f32)[0]

        mr = mgt_r_ref[:, cs]
        gmat = mr == gi
        gcls = to_col(gcls_r_ref[:, cs]).astype(f32)
        tcls_out[b, :] = jnp.sum(jnp.where(gmat, gcls, 0.0), axis=0)


def _tc_b_body(mcls_ref, q_ref, l1_ref, cnt_ref, base_ref, win_ref, tcls_ref,
               cls_out, conf_out, coord_out):
    f32 = jnp.float32

    corr = f32(0.0)
    for b in range(B):
        cs = pl.ds(b * G, G)
        winner = win_ref[b:b + 1, :]
        tcls = tcls_ref[b:b + 1, :]
        xm = mcls_ref[:, cs]
        lsoft = xm - _lse0(xm)
        lane0 = lax.broadcasted_iota(jnp.int32, (NUM_CLASSES, G), 0)
        onehot = (lane0.astype(f32) == tcls).astype(f32)
        logp_t = jnp.sum(lsoft * onehot, axis=0, keepdims=True)
        p_t = jnp.exp(logp_t)
        alpha = jnp.where(tcls == 0.0, ALPHA_BG, 1.0 - ALPHA_BG)
        loss_new = -alpha * (1.0 - p_t) * (1.0 - p_t) * logp_t
        ls4m = lsoft[BACKGROUND:BACKGROUND + 1, :]
        p4m = jnp.exp(ls4m)
        loss_old = -(1.0 - ALPHA_BG) * (1.0 - p4m) * (1.0 - p4m) * ls4m
        corr = corr + jnp.sum(winner * (loss_new - loss_old))

    class_loss = CLASS_W * (base_ref[0, 0] + corr) / f32(N_ROW)

    conf_loss = PT_CONF_W * (-jnp.sum(jnp.log(q_ref[...])) / f32(N_PAIR * P))

    l1s = jnp.sum(l1_ref[...])
    cnts = jnp.sum(cnt_ref[...])
    coord_loss = PT_COORD_W * l1s / jnp.maximum(cnts, 1.0)

    cls_out[...] = jnp.reshape(class_loss, (1, 1))
    conf_out[...] = jnp.reshape(conf_loss, (1, 1))
    coord_out[...] = jnp.reshape(coord_loss, (1, 1))


def _tc_prep(cls_t, src_row, mgt_row, gcls_row):
    return pl.pallas_call(
        _tc_a_body,
        out_shape=(jax.ShapeDtypeStruct((1, 1), jnp.float32),
                   jax.ShapeDtypeStruct((B, G), jnp.float32),
                   jax.ShapeDtypeStruct((B, G), jnp.float32)),
    )(cls_t, src_row, mgt_row, gcls_row)


def _tc_losses(mcls, q, l1_part, cnt_part, base, win, tcls):
    s = jax.ShapeDtypeStruct((1, 1), jnp.float32)
    return pl.pallas_call(
        _tc_b_body,
        out_shape=(s, s, s),
    )(mcls, q, l1_part, cnt_part, base, win, tcls)


def kernel(cls_pred, point_coord_pred, point_confidence_pred,
           matched_src_idx, matched_gt_idx, gt_class, gt_points,
           gt_pt_padding_flags, gt_num):
    i32 = jnp.int32
    srcm = matched_src_idx.astype(i32)
    mgtm = matched_gt_idx.astype(i32)

    conf_t = jnp.transpose(point_confidence_pred, (0, 2, 1))
    coord_t = jnp.transpose(point_coord_pred, (0, 2, 3, 1))
    gtpt_t = jnp.transpose(gt_points, (1, 2, 0))
    flags_t = jnp.transpose(gt_pt_padding_flags.astype(i32), (1, 0))
    cls_t = jnp.transpose(cls_pred, (2, 0, 1))

    q, mcls, l1_part, cnt_part = _sc_assemble(
        srcm, mgtm, conf_t, coord_t, gtpt_t, flags_t, cls_t)

    base, win, tcls = _tc_prep(cls_t, srcm.reshape(1, N_PAIR),
                               mgtm.reshape(1, N_PAIR),
                               gt_class.astype(i32).reshape(1, N_PAIR))

    cl, co, cd = _tc_losses(mcls, q, l1_part.reshape(4, 128),
                            cnt_part.reshape(4, 128), base, win, tcls)
    return (cl.reshape(()), co.reshape(()), cd.reshape(()))


# speedup vs baseline: 1.3741x; 1.0105x over previous
import functools

import jax
import jax.numpy as jnp
from jax import lax
from jax.experimental import pallas as pl
from jax.experimental.pallas import tpu as pltpu
from jax.experimental.pallas import tpu_sc as plsc

NUM_CLASSES = 5
BACKGROUND = 4
GAMMA = 2.0
ALPHA_BG = 0.25
CLASS_W = 2.0
PT_CONF_W = 1.0
PT_COORD_W = 5.0
PAD_VALUE = -10000.0

B, Q, G, P = 8, 512, 128, 64
N_PAIR = B * G
N_ROW = B * Q


def _sc_assemble(srcm, mgtm, conf_t, coord_t, gtpt_t, flags_t, cls_t):
    info = plsc.get_sparse_core_info()
    nc, ns = info.num_cores, info.num_subcores
    nw = nc * ns
    pts = P // (nw // B)
    mesh = plsc.VectorSubcoreMesh(core_axis_name="c", subcore_axis_name="s")

    @functools.partial(
        pl.kernel,
        mesh=mesh,
        compiler_params=pltpu.CompilerParams(needs_layout_passes=False,
                                             skip_device_barrier=True),
        out_type=[
            jax.ShapeDtypeStruct((P, N_PAIR), jnp.float32),
            jax.ShapeDtypeStruct((NUM_CLASSES, N_PAIR), jnp.float32),
            jax.ShapeDtypeStruct((nw * 16,), jnp.float32),
            jax.ShapeDtypeStruct((nw * 16,), jnp.float32),
        ],
        scratch_types=[
            pltpu.VMEM((B, G), jnp.int32),
            pltpu.VMEM((B, G), jnp.int32),
            pltpu.VMEM((pts, Q), jnp.float32),
            pltpu.VMEM((pts, G), jnp.int32),
            pltpu.VMEM((pts, 2, Q), jnp.float32),
            pltpu.VMEM((pts, 2, G), jnp.float32),
            pltpu.VMEM((NUM_CLASSES, Q), jnp.float32),
            pltpu.VMEM((pts, G), jnp.float32),
            pltpu.VMEM((NUM_CLASSES, G), jnp.float32),
            pltpu.VMEM((16,), jnp.float32),
            pltpu.VMEM((16,), jnp.float32),
            pltpu.SemaphoreType.DMA,
            pltpu.SemaphoreType.DMA,
        ],
    )
    def sc_kernel(src_hbm, mgt_hbm, conf_hbm, coord_hbm, gtpt_hbm, flags_hbm,
                  cls_hbm, q_out, mcls_out, l1_out, cnt_out,
                  src_all, mgt_all, conf_sl, flag_sl, coord_sl, gtpt_sl,
                  cls_sl, q_sl, mcls_sl, l1_v, cnt_v, sem, osem):
        wid = lax.axis_index("s") * nc + lax.axis_index("c")
        b = wid // (nw // B)
        p0 = (wid % (nw // B)) * pts

        pend = [
            pltpu.async_copy(src_hbm, src_all, sem),
            pltpu.async_copy(mgt_hbm, mgt_all, sem),
            pltpu.async_copy(conf_hbm.at[b, pl.ds(p0, pts)], conf_sl, sem),
            pltpu.async_copy(
                flags_hbm.at[pl.ds(p0, pts), pl.ds(b * G, G)], flag_sl, sem),
            pltpu.async_copy(coord_hbm.at[b, pl.ds(p0, pts)], coord_sl, sem),
            pltpu.async_copy(
                gtpt_hbm.at[pl.ds(p0, pts), :, pl.ds(b * G, G)], gtpt_sl,
                sem),
        ]

        @pl.when(wid < B)
        def _():
            pltpu.async_copy(cls_hbm.at[:, wid % B], cls_sl, sem).wait()
        for cp in pend:
            cp.wait()

        lane16 = lax.broadcasted_iota(jnp.int32, (16,), 0)

        opend = []
        def chunk(k, carry):
            col = lane16 + k * 16
            bv = jnp.full((16,), b, jnp.int32)
            idx = plsc.load_gather(src_all, [bv, col])
            gti = plsc.load_gather(mgt_all, [bv, col])

            def prow(i, carry2):
                l1, cnt = carry2
                row = jnp.full((16,), 0, jnp.int32) + i
                pv = plsc.load_gather(conf_sl, [row, idx])
                fv = plsc.load_gather(flag_sl, [row, gti])
                plsc.store_scatter(q_sl, [row, col],
                                   jnp.where(fv != 0, pv, 1.0 - pv))
                for c in range(2):
                    cv = jnp.full((16,), c, jnp.int32)
                    sp = plsc.load_gather(coord_sl, [row, cv, idx])
                    tp = plsc.load_gather(gtpt_sl, [row, cv, gti])
                    m = jnp.where(tp != PAD_VALUE, 1.0, 0.0)
                    l1 = l1 + jnp.abs(sp - tp) * m
                    cnt = cnt + m
                return (l1, cnt)

            return lax.fori_loop(0, pts, prow, carry)

        l1, cnt = lax.fori_loop(0, G // 16, chunk,
                                (jnp.zeros((16,), jnp.float32),
                                 jnp.zeros((16,), jnp.float32)))
        opend.append(pltpu.async_copy(
            q_sl, q_out.at[pl.ds(p0, pts), pl.ds(b * G, G)], osem))

        @pl.when(wid < B)
        def _():
            cb = wid % B

            def mchunk(k, _):
                col = lane16 + k * 16
                idx = plsc.load_gather(
                    src_all, [jnp.full((16,), cb, jnp.int32), col])
                for cc in range(NUM_CLASSES):
                    ccv = jnp.full((16,), cc, jnp.int32)
                    plsc.store_scatter(
                        mcls_sl, [ccv, col],
                        plsc.load_gather(cls_sl, [ccv, idx]))
                return 0

            lax.fori_loop(0, G // 16, mchunk, 0)
            pltpu.async_copy(
                mcls_sl, mcls_out.at[:, pl.ds(cb * G, G)], osem).wait()

        l1_v[...] = l1
        cnt_v[...] = cnt
        opend.append(pltpu.async_copy(l1_v, l1_out.at[pl.ds(wid * 16, 16)],
                                      osem))
        opend.append(pltpu.async_copy(cnt_v, cnt_out.at[pl.ds(wid * 16, 16)],
                                      osem))
        for cp in opend:
            cp.wait()

    return sc_kernel(srcm, mgtm, conf_t, coord_t, gtpt_t, flags_t, cls_t)


def _lse0(x):
    m = jnp.max(x, axis=0, keepdims=True)
    return jnp.log(jnp.sum(jnp.exp(x - m), axis=0, keepdims=True)) + m


def _tc_a_body(cls_ref, src_r_ref, mgt_r_ref, gcls_r_ref,
               base_out, win_out, tcls_out):
    f32 = jnp.float32

    eye = (lax.broadcasted_iota(jnp.int32, (G, G), 0)
           == lax.broadcasted_iota(jnp.int32, (G, G), 1))

    def to_col(vrow):
        return jnp.sum(jnp.where(eye, vrow, 0), axis=1, keepdims=True)

    x = cls_ref[...]
    ls4 = x[BACKGROUND:BACKGROUND + 1] - _lse0(x)
    p4 = jnp.exp(ls4)
    base_sum = jnp.sum(-(1.0 - ALPHA_BG) * (1.0 - p4) * (1.0 - p4) * ls4)
    base_out[...] = jnp.reshape(base_sum, (1, 1))

    gi = lax.broadcasted_iota(jnp.int32, (G, G), 0)
    gj = lax.broadcasted_iota(jnp.int32, (G, G), 1)
    for b in range(B):
        cs = pl.ds(b * G, G)
        sr = src_r_ref[:, cs]
        sc = to_col(sr)
        later = jnp.where((sc == sr) & (gi > gj), 1, 0)
        conflict = jnp.max(later, axis=0, keepdims=True)
        win_out[b, :] = (conflict == 0).astype(f32)[0]

        mr = mgt_r_ref[:, cs]
        gmat = mr == gi
        gcls = to_col(gcls_r_ref[:, cs]).astype(f32)
        tcls_out[b, :] = jnp.sum(jnp.where(gmat, gcls, 0.0), axis=0)


def _tc_b_body(mcls_ref, q_ref, l1_ref, cnt_ref, base_ref, win_ref, tcls_ref,
               cls_out, conf_out, coord_out):
    f32 = jnp.float32

    corr = f32(0.0)
    for b in range(B):
        cs = pl.ds(b * G, G)
        winner = win_ref[b:b + 1, :]
        tcls = tcls_ref[b:b + 1, :]
        xm = mcls_ref[:, cs]
        lsoft = xm - _lse0(xm)
        lane0 = lax.broadcasted_iota(jnp.int32, (NUM_CLASSES, G), 0)
        onehot = (lane0.astype(f32) == tcls).astype(f32)
        logp_t = jnp.sum(lsoft * onehot, axis=0, keepdims=True)
        p_t = jnp.exp(logp_t)
        alpha = jnp.where(tcls == 0.0, ALPHA_BG, 1.0 - ALPHA_BG)
        loss_new = -alpha * (1.0 - p_t) * (1.0 - p_t) * logp_t
        ls4m = lsoft[BACKGROUND:BACKGROUND + 1, :]
        p4m = jnp.exp(ls4m)
        loss_old = -(1.0 - ALPHA_BG) * (1.0 - p4m) * (1.0 - p4m) * ls4m
        corr = corr + jnp.sum(winner * (loss_new - loss_old))

    class_loss = CLASS_W * (base_ref[0, 0] + corr) / f32(N_ROW)

    conf_loss = PT_CONF_W * (-jnp.sum(jnp.log(q_ref[...])) / f32(N_PAIR * P))

    l1s = jnp.sum(l1_ref[...])
    cnts = jnp.sum(cnt_ref[...])
    coord_loss = PT_COORD_W * l1s / jnp.maximum(cnts, 1.0)

    cls_out[...] = jnp.reshape(class_loss, (1, 1))
    conf_out[...] = jnp.reshape(conf_loss, (1, 1))
    coord_out[...] = jnp.reshape(coord_loss, (1, 1))


def _tc_prep(cls_t, src_row, mgt_row, gcls_row):
    return pl.pallas_call(
        _tc_a_body,
        out_shape=(jax.ShapeDtypeStruct((1, 1), jnp.float32),
                   jax.ShapeDtypeStruct((B, G), jnp.float32),
                   jax.ShapeDtypeStruct((B, G), jnp.float32)),
    )(cls_t, src_row, mgt_row, gcls_row)


def _tc_losses(mcls, q, l1_part, cnt_part, base, win, tcls):
    s = jax.ShapeDtypeStruct((1, 1), jnp.float32)
    return pl.pallas_call(
        _tc_b_body,
        out_shape=(s, s, s),
    )(mcls, q, l1_part, cnt_part, base, win, tcls)


def kernel(cls_pred, point_coord_pred, point_confidence_pred,
           matched_src_idx, matched_gt_idx, gt_class, gt_points,
           gt_pt_padding_flags, gt_num):
    i32 = jnp.int32
    srcm = matched_src_idx.astype(i32)
    mgtm = matched_gt_idx.astype(i32)

    conf_t = jnp.transpose(point_confidence_pred, (0, 2, 1))
    coord_t = jnp.transpose(point_coord_pred, (0, 2, 3, 1))
    gtpt_t = jnp.transpose(gt_points, (1, 2, 0))
    flags_t = jnp.transpose(gt_pt_padding_flags.astype(i32), (1, 0))
    cls_t = jnp.transpose(cls_pred, (2, 0, 1))

    q, mcls, l1_part, cnt_part = _sc_assemble(
        srcm, mgtm, conf_t, coord_t, gtpt_t, flags_t, cls_t)

    base, win, tcls = _tc_prep(cls_t, srcm.reshape(1, N_PAIR),
                               mgtm.reshape(1, N_PAIR),
                               gt_class.astype(i32).reshape(1, N_PAIR))

    cl, co, cd = _tc_losses(mcls, q, l1_part.reshape(4, 128),
                            cnt_part.reshape(4, 128), base, win, tcls)
    return (cl.reshape(()), co.reshape(()), cd.reshape(()))


# speedup vs baseline: 1.3782x; 1.0030x over previous
import functools

import jax
import jax.numpy as jnp
from jax import lax
from jax.experimental import pallas as pl
from jax.experimental.pallas import tpu as pltpu
from jax.experimental.pallas import tpu_sc as plsc

NUM_CLASSES = 5
BACKGROUND = 4
GAMMA = 2.0
ALPHA_BG = 0.25
CLASS_W = 2.0
PT_CONF_W = 1.0
PT_COORD_W = 5.0
PAD_VALUE = -10000.0

B, Q, G, P = 8, 512, 128, 64
N_PAIR = B * G
N_ROW = B * Q


def _sc_assemble(srcm, mgtm, conf_t, coord_t, gtpt_t, flags_t, cls_t):
    info = plsc.get_sparse_core_info()
    nc, ns = info.num_cores, info.num_subcores
    nw = nc * ns
    pts = P // (nw // B)
    mesh = plsc.VectorSubcoreMesh(core_axis_name="c", subcore_axis_name="s")

    @functools.partial(
        pl.kernel,
        mesh=mesh,
        compiler_params=pltpu.CompilerParams(needs_layout_passes=False,
                                             skip_device_barrier=True),
        out_type=[
            jax.ShapeDtypeStruct((P, N_PAIR), jnp.float32),
            jax.ShapeDtypeStruct((NUM_CLASSES, N_PAIR), jnp.float32),
            jax.ShapeDtypeStruct((nw * 16,), jnp.float32),
            jax.ShapeDtypeStruct((nw * 16,), jnp.float32),
        ],
        scratch_types=[
            pltpu.VMEM((B, G), jnp.int32),
            pltpu.VMEM((B, G), jnp.int32),
            pltpu.VMEM((pts, Q), jnp.float32),
            pltpu.VMEM((pts, G), jnp.int32),
            pltpu.VMEM((pts, 2, Q), jnp.float32),
            pltpu.VMEM((pts, 2, G), jnp.float32),
            pltpu.VMEM((NUM_CLASSES, Q), jnp.float32),
            pltpu.VMEM((pts, G), jnp.float32),
            pltpu.VMEM((NUM_CLASSES, G), jnp.float32),
            pltpu.VMEM((16,), jnp.float32),
            pltpu.VMEM((16,), jnp.float32),
            pltpu.SemaphoreType.DMA,
            pltpu.SemaphoreType.DMA,
        ],
    )
    def sc_kernel(src_hbm, mgt_hbm, conf_hbm, coord_hbm, gtpt_hbm, flags_hbm,
                  cls_hbm, q_out, mcls_out, l1_out, cnt_out,
                  src_all, mgt_all, conf_sl, flag_sl, coord_sl, gtpt_sl,
                  cls_sl, q_sl, mcls_sl, l1_v, cnt_v, sem, osem):
        wid = lax.axis_index("s") * nc + lax.axis_index("c")
        b = wid // (nw // B)
        p0 = (wid % (nw // B)) * pts

        pend = [
            pltpu.async_copy(src_hbm, src_all, sem),
            pltpu.async_copy(mgt_hbm, mgt_all, sem),
            pltpu.async_copy(conf_hbm.at[b, pl.ds(p0, pts)], conf_sl, sem),
            pltpu.async_copy(
                flags_hbm.at[pl.ds(p0, pts), pl.ds(b * G, G)], flag_sl, sem),
            pltpu.async_copy(coord_hbm.at[b, pl.ds(p0, pts)], coord_sl, sem),
            pltpu.async_copy(
                gtpt_hbm.at[pl.ds(p0, pts), :, pl.ds(b * G, G)], gtpt_sl,
                sem),
        ]

        @pl.when(wid < B)
        def _():
            pltpu.async_copy(cls_hbm.at[:, wid % B], cls_sl, sem).wait()
        for cp in pend:
            cp.wait()

        lane16 = lax.broadcasted_iota(jnp.int32, (16,), 0)

        opend = []
        def chunk(k, carry):
            col = lane16 + k * 16
            bv = jnp.full((16,), b, jnp.int32)
            idx = plsc.load_gather(src_all, [bv, col])
            gti = plsc.load_gather(mgt_all, [bv, col])

            def prow(i, carry2):
                l1, cnt = carry2
                row = jnp.full((16,), 0, jnp.int32) + i
                pv = plsc.load_gather(conf_sl, [row, idx])
                fv = plsc.load_gather(flag_sl, [row, gti])
                plsc.store_scatter(q_sl, [row, col],
                                   jnp.where(fv != 0, pv, 1.0 - pv))
                for c in range(2):
                    cv = jnp.full((16,), c, jnp.int32)
                    sp = plsc.load_gather(coord_sl, [row, cv, idx])
                    tp = plsc.load_gather(gtpt_sl, [row, cv, gti])
                    m = jnp.where(tp != PAD_VALUE, 1.0, 0.0)
                    l1 = l1 + jnp.abs(sp - tp) * m
                    cnt = cnt + m
                return (l1, cnt)

            return lax.fori_loop(0, pts, prow, carry, unroll=2)

        l1, cnt = lax.fori_loop(0, G // 16, chunk,
                                (jnp.zeros((16,), jnp.float32),
                                 jnp.zeros((16,), jnp.float32)))
        opend.append(pltpu.async_copy(
            q_sl, q_out.at[pl.ds(p0, pts), pl.ds(b * G, G)], osem))

        @pl.when(wid < B)
        def _():
            cb = wid % B

            def mchunk(k, _):
                col = lane16 + k * 16
                idx = plsc.load_gather(
                    src_all, [jnp.full((16,), cb, jnp.int32), col])
                for cc in range(NUM_CLASSES):
                    ccv = jnp.full((16,), cc, jnp.int32)
                    plsc.store_scatter(
                        mcls_sl, [ccv, col],
                        plsc.load_gather(cls_sl, [ccv, idx]))
                return 0

            lax.fori_loop(0, G // 16, mchunk, 0)
            pltpu.async_copy(
                mcls_sl, mcls_out.at[:, pl.ds(cb * G, G)], osem).wait()

        l1_v[...] = l1
        cnt_v[...] = cnt
        opend.append(pltpu.async_copy(l1_v, l1_out.at[pl.ds(wid * 16, 16)],
                                      osem))
        opend.append(pltpu.async_copy(cnt_v, cnt_out.at[pl.ds(wid * 16, 16)],
                                      osem))
        for cp in opend:
            cp.wait()

    return sc_kernel(srcm, mgtm, conf_t, coord_t, gtpt_t, flags_t, cls_t)


def _lse0(x):
    m = jnp.max(x, axis=0, keepdims=True)
    return jnp.log(jnp.sum(jnp.exp(x - m), axis=0, keepdims=True)) + m


def _tc_a_body(cls_ref, src_r_ref, mgt_r_ref, gcls_r_ref,
               base_out, win_out, tcls_out):
    f32 = jnp.float32

    eye = (lax.broadcasted_iota(jnp.int32, (G, G), 0)
           == lax.broadcasted_iota(jnp.int32, (G, G), 1))

    def to_col(vrow):
        return jnp.sum(jnp.where(eye, vrow, 0), axis=1, keepdims=True)

    x = cls_ref[...]
    ls4 = x[BACKGROUND:BACKGROUND + 1] - _lse0(x)
    p4 = jnp.exp(ls4)
    base_sum = jnp.sum(-(1.0 - ALPHA_BG) * (1.0 - p4) * (1.0 - p4) * ls4)
    base_out[...] = jnp.reshape(base_sum, (1, 1))

    gi = lax.broadcasted_iota(jnp.int32, (G, G), 0)
    gj = lax.broadcasted_iota(jnp.int32, (G, G), 1)
    for b in range(B):
        cs = pl.ds(b * G, G)
        sr = src_r_ref[:, cs]
        sc = to_col(sr)
        later = jnp.where((sc == sr) & (gi > gj), 1, 0)
        conflict = jnp.max(later, axis=0, keepdims=True)
        win_out[b, :] = (conflict == 0).astype(f32)[0]

        mr = mgt_r_ref[:, cs]
        gmat = mr == gi
        gcls = to_col(gcls_r_ref[:, cs]).astype(f32)
        tcls_out[b, :] = jnp.sum(jnp.where(gmat, gcls, 0.0), axis=0)


def _tc_b_body(mcls_ref, q_ref, l1_ref, cnt_ref, base_ref, win_ref, tcls_ref,
               cls_out, conf_out, coord_out):
    f32 = jnp.float32

    corr = f32(0.0)
    for b in range(B):
        cs = pl.ds(b * G, G)
        winner = win_ref[b:b + 1, :]
        tcls = tcls_ref[b:b + 1, :]
        xm = mcls_ref[:, cs]
        lsoft = xm - _lse0(xm)
        lane0 = lax.broadcasted_iota(jnp.int32, (NUM_CLASSES, G), 0)
        onehot = (lane0.astype(f32) == tcls).astype(f32)
        logp_t = jnp.sum(lsoft * onehot, axis=0, keepdims=True)
        p_t = jnp.exp(logp_t)
        alpha = jnp.where(tcls == 0.0, ALPHA_BG, 1.0 - ALPHA_BG)
        loss_new = -alpha * (1.0 - p_t) * (1.0 - p_t) * logp_t
        ls4m = lsoft[BACKGROUND:BACKGROUND + 1, :]
        p4m = jnp.exp(ls4m)
        loss_old = -(1.0 - ALPHA_BG) * (1.0 - p4m) * (1.0 - p4m) * ls4m
        corr = corr + jnp.sum(winner * (loss_new - loss_old))

    class_loss = CLASS_W * (base_ref[0, 0] + corr) / f32(N_ROW)

    conf_loss = PT_CONF_W * (-jnp.sum(jnp.log(q_ref[...])) / f32(N_PAIR * P))

    l1s = jnp.sum(l1_ref[...])
    cnts = jnp.sum(cnt_ref[...])
    coord_loss = PT_COORD_W * l1s / jnp.maximum(cnts, 1.0)

    cls_out[...] = jnp.reshape(class_loss, (1, 1))
    conf_out[...] = jnp.reshape(conf_loss, (1, 1))
    coord_out[...] = jnp.reshape(coord_loss, (1, 1))


def _tc_prep(cls_t, src_row, mgt_row, gcls_row):
    return pl.pallas_call(
        _tc_a_body,
        out_shape=(jax.ShapeDtypeStruct((1, 1), jnp.float32),
                   jax.ShapeDtypeStruct((B, G), jnp.float32),
                   jax.ShapeDtypeStruct((B, G), jnp.float32)),
    )(cls_t, src_row, mgt_row, gcls_row)


def _tc_losses(mcls, q, l1_part, cnt_part, base, win, tcls):
    s = jax.ShapeDtypeStruct((1, 1), jnp.float32)
    return pl.pallas_call(
        _tc_b_body,
        out_shape=(s, s, s),
    )(mcls, q, l1_part, cnt_part, base, win, tcls)


def kernel(cls_pred, point_coord_pred, point_confidence_pred,
           matched_src_idx, matched_gt_idx, gt_class, gt_points,
           gt_pt_padding_flags, gt_num):
    i32 = jnp.int32
    srcm = matched_src_idx.astype(i32)
    mgtm = matched_gt_idx.astype(i32)

    conf_t = jnp.transpose(point_confidence_pred, (0, 2, 1))
    coord_t = jnp.transpose(point_coord_pred, (0, 2, 3, 1))
    gtpt_t = jnp.transpose(gt_points, (1, 2, 0))
    flags_t = jnp.transpose(gt_pt_padding_flags.astype(i32), (1, 0))
    cls_t = jnp.transpose(cls_pred, (2, 0, 1))

    q, mcls, l1_part, cnt_part = _sc_assemble(
        srcm, mgtm, conf_t, coord_t, gtpt_t, flags_t, cls_t)

    base, win, tcls = _tc_prep(cls_t, srcm.reshape(1, N_PAIR),
                               mgtm.reshape(1, N_PAIR),
                               gt_class.astype(i32).reshape(1, N_PAIR))

    cl, co, cd = _tc_losses(mcls, q, l1_part.reshape(4, 128),
                            cnt_part.reshape(4, 128), base, win, tcls)
    return (cl.reshape(()), co.reshape(()), cd.reshape(()))
